# baseline (speedup 1.0000x reference)
import functools
import math

import jax
import jax.numpy as jnp
from jax import lax
from jax.experimental import pallas as pl
from jax.experimental.pallas import tpu as pltpu
from jax.experimental.pallas import tpu_sc as plsc

VOCAB = 1000000
MAX_POS = 512
HIDDEN = 64
BATCH = 4096
SEQ = 50

NC = 2
NS = 16
NW = NC * NS
LANES = 16

TOTAL_ROWS = BATCH * SEQ
ROWS_PER_W = TOTAL_ROWS // NW
SEQS_PER_CHUNK = 16
CHUNK = SEQS_PER_CHUNK * SEQ
NCHUNK = ROWS_PER_W // CHUNK
VPR = HIDDEN // LANES
SEQ_PAD = 56


def _emb_kernel(ids_hbm, table_hbm, pos_hbm, out_hbm,
                idx_v, rows_v, pos_v, sem):
    wid = lax.axis_index("s") * NC + lax.axis_index("c")
    wbase = wid * ROWS_PER_W

    pltpu.sync_copy(pos_hbm.at[pl.ds(0, SEQ_PAD)], pos_v)
    scale = jnp.float32(1.0 / math.sqrt(HIDDEN))

    def _scale_row(r, _):
        for q in range(VPR):
            pos_v[r, pl.ds(q * LANES, LANES)] = (
                pos_v[r, pl.ds(q * LANES, LANES)] * scale)
        return 0

    lax.fori_loop(0, SEQ, _scale_row, 0)

    def _chunk(c, _):
        base = wbase + c * CHUNK
        pltpu.sync_copy(ids_hbm.at[pl.ds(base, CHUNK)], idx_v)
        pltpu.async_copy(table_hbm.at[idx_v], rows_v, sem).wait()

        def _seq(s, _):
            row0 = s * SEQ
            for r in range(SEQ):
                for q in range(VPR):
                    sl = pl.ds(q * LANES, LANES)
                    rows_v[row0 + r, sl] = rows_v[row0 + r, sl] + pos_v[r, sl]
            return 0

        lax.fori_loop(0, SEQS_PER_CHUNK, _seq, 0)
        pltpu.sync_copy(rows_v, out_hbm.at[pl.ds(base, CHUNK)])
        return 0

    lax.fori_loop(0, NCHUNK, _chunk, 0)


@jax.jit
def _emb(ids_flat, ids_table, pos_table):
    mesh = plsc.VectorSubcoreMesh(core_axis_name="c", subcore_axis_name="s")
    f = pl.kernel(
        _emb_kernel,
        out_type=jax.ShapeDtypeStruct((TOTAL_ROWS, HIDDEN), jnp.float32),
        mesh=mesh,
        scratch_types=[
            pltpu.VMEM((CHUNK,), jnp.int32),
            pltpu.VMEM((CHUNK, HIDDEN), jnp.float32),
            pltpu.VMEM((SEQ_PAD, HIDDEN), jnp.float32),
            pltpu.SemaphoreType.DMA,
        ],
        compiler_params=pltpu.CompilerParams(use_tc_tiling_on_sc=False),
    )
    return f(ids_flat, ids_table, pos_table)


def kernel(input_ids, ids_table, pos_table):
    ids_flat = input_ids.reshape(-1)
    out = _emb(ids_flat, ids_table, pos_table)
    return out.reshape(BATCH, SEQ, HIDDEN)


# speedup vs baseline: 1.0099x; 1.0099x over previous
import functools
import math

import jax
import jax.numpy as jnp
from jax import lax
from jax.experimental import pallas as pl
from jax.experimental.pallas import tpu as pltpu
from jax.experimental.pallas import tpu_sc as plsc

VOCAB = 1000000
MAX_POS = 512
HIDDEN = 64
BATCH = 4096
SEQ = 50

NC = 2
NS = 16
NW = NC * NS
LANES = 16

TOTAL_ROWS = BATCH * SEQ
ROWS_PER_W = TOTAL_ROWS // NW
SEQS_PER_CHUNK = 16
CHUNK = SEQS_PER_CHUNK * SEQ
NCHUNK = ROWS_PER_W // CHUNK
VPR = HIDDEN // LANES
SEQ_PAD = 56
RBLOCK = 25
NBLOCK = CHUNK // RBLOCK


def _emb_kernel(ids_hbm, table_hbm, pos_hbm, out_hbm,
                idx0, idx1, rows0, rows1, pos_v,
                gsem0, gsem1, wsem0, wsem1):
    wid = lax.axis_index("s") * NC + lax.axis_index("c")
    wbase = wid * ROWS_PER_W

    idx_v = [idx0, idx1]
    rows_v = [rows0, rows1]
    gsem = [gsem0, gsem1]
    wsem = [wsem0, wsem1]

    pltpu.sync_copy(pos_hbm.at[pl.ds(0, SEQ_PAD)], pos_v)
    scale = jnp.float32(1.0 / math.sqrt(HIDDEN))

    def _scale_row(r, _):
        for q in range(VPR):
            pos_v[r, pl.ds(q * LANES, LANES)] = (
                pos_v[r, pl.ds(q * LANES, LANES)] * scale)
        return 0

    lax.fori_loop(0, SEQ, _scale_row, 0)

    def _add_pos(buf):
        def _block(b, _):
            row0 = b * RBLOCK
            p0 = (b % 2) * RBLOCK
            for r in range(RBLOCK):
                for q in range(VPR):
                    sl = pl.ds(q * LANES, LANES)
                    buf[row0 + r, sl] = buf[row0 + r, sl] + pos_v[p0 + r, sl]
            return 0

        lax.fori_loop(0, NBLOCK, _block, 0)

    def _start_gather(c):
        b = c % 2
        base = wbase + c * CHUNK
        pltpu.sync_copy(ids_hbm.at[pl.ds(base, CHUNK)], idx_v[b])
        return pltpu.async_copy(table_hbm.at[idx_v[b]], rows_v[b], gsem[b])

    writes = [None, None]
    gathers = [None, None]
    gathers[0] = _start_gather(0)
    for c in range(NCHUNK):
        b = c % 2
        nb = (c + 1) % 2
        if c + 1 < NCHUNK:
            if writes[nb] is not None:
                writes[nb].wait()
                writes[nb] = None
            gathers[nb] = _start_gather(c + 1)
        gathers[b].wait()
        _add_pos(rows_v[b])
        base = wbase + c * CHUNK
        writes[b] = pltpu.async_copy(
            rows_v[b], out_hbm.at[pl.ds(base, CHUNK)], wsem[b])
    for w in writes:
        if w is not None:
            w.wait()


@jax.jit
def _emb(ids_flat, ids_table, pos_table):
    mesh = plsc.VectorSubcoreMesh(core_axis_name="c", subcore_axis_name="s")
    f = pl.kernel(
        _emb_kernel,
        out_type=jax.ShapeDtypeStruct((TOTAL_ROWS, HIDDEN), jnp.float32),
        mesh=mesh,
        scratch_types=[
            pltpu.VMEM((CHUNK,), jnp.int32),
            pltpu.VMEM((CHUNK,), jnp.int32),
            pltpu.VMEM((CHUNK, HIDDEN), jnp.float32),
            pltpu.VMEM((CHUNK, HIDDEN), jnp.float32),
            pltpu.VMEM((SEQ_PAD, HIDDEN), jnp.float32),
            pltpu.SemaphoreType.DMA,
            pltpu.SemaphoreType.DMA,
            pltpu.SemaphoreType.DMA,
            pltpu.SemaphoreType.DMA,
        ],
        compiler_params=pltpu.CompilerParams(use_tc_tiling_on_sc=False),
    )
    return f(ids_flat, ids_table, pos_table)


def kernel(input_ids, ids_table, pos_table):
    ids_flat = input_ids.reshape(-1)
    out = _emb(ids_flat, ids_table, pos_table)
    return out.reshape(BATCH, SEQ, HIDDEN)
